# baseline (speedup 1.0000x reference)
import functools

import jax
import jax.numpy as jnp
from jax.experimental import pallas as pl
from jax.experimental.pallas import tpu as pltpu
from jax.experimental.pallas import tpu_sc as plsc

QN = 1024
KD = 16
KN = 100000
VD = 978
NK = 5

KPAD = 102400
BK = 2048
NBLK = KPAD // BK

FMAX = float(3.402823466e38)
IMAX = 2**31 - 1

GW = 16
VDP = 1024


def _topk_body(q_ref, kb_ref, vals_ref, idx_ref):
    b = pl.program_id(0)

    @pl.when(b == 0)
    def _init():
        vals_ref[...] = jnp.full((QN, 16), FMAX, jnp.float32)
        idx_ref[...] = jnp.full((QN, 16), IMAX, jnp.int32)

    q = q_ref[...]
    kb = kb_ref[...]
    qk = jax.lax.dot_general(
        q, kb, (((1,), (1,)), ((), ())),
        preferred_element_type=jnp.float32)
    qsq = jnp.sum(q * q, axis=1, keepdims=True)
    ksq = jnp.sum(kb * kb, axis=1)
    d2 = (qsq - 2.0 * qk) + ksq[None, :]
    gidx = b * BK + jax.lax.broadcasted_iota(jnp.int32, (QN, BK), 1)
    d2 = jnp.where(gidx >= KN, FMAX, d2)

    s = d2
    for j in range(NK):
        m = jnp.min(s, axis=1, keepdims=True)
        am = jnp.min(jnp.where(s == m, gidx, IMAX), axis=1,
                     keepdims=True)
        vals_ref[:, 8 + j:9 + j] = m
        idx_ref[:, 8 + j:9 + j] = am
        if j + 1 < NK:
            s = jnp.where(gidx == am, FMAX, s)

    cv = vals_ref[...]
    ci = idx_ref[...]
    nv = []
    ni = []
    for j in range(NK):
        m = jnp.min(cv, axis=1, keepdims=True)
        am = jnp.min(jnp.where(cv == m, ci, IMAX), axis=1, keepdims=True)
        nv.append(m)
        ni.append(am)
        if j + 1 < NK:
            cv = jnp.where(ci == am, FMAX, cv)
    vals_ref[:, 0:NK] = jnp.concatenate(nv, axis=1)
    idx_ref[:, 0:NK] = jnp.concatenate(ni, axis=1)


@jax.jit
def _topk(queries, keys_padded):
    vals, idx = pl.pallas_call(
        _topk_body,
        grid=(NBLK,),
        in_specs=[
            pl.BlockSpec((QN, KD), lambda i: (0, 0)),
            pl.BlockSpec((BK, KD), lambda i: (i, 0)),
        ],
        out_specs=[
            pl.BlockSpec((QN, 16), lambda i: (0, 0)),
            pl.BlockSpec((QN, 16), lambda i: (0, 0)),
        ],
        out_shape=[
            jax.ShapeDtypeStruct((QN, 16), jnp.float32),
            jax.ShapeDtypeStruct((QN, 16), jnp.int32),
        ],
        compiler_params=pltpu.CompilerParams(
            dimension_semantics=("arbitrary",)),
    )(queries, keys_padded)
    del vals
    return idx


def _sc_gather(values, flat_idx):
    mesh = plsc.VectorSubcoreMesh(core_axis_name="core",
                                  subcore_axis_name="subcore")

    @pl.kernel(out_type=jax.ShapeDtypeStruct((NK * QN, VDP), values.dtype),
               mesh=mesh)
    def kern(x_hbm, i_hbm, o_hbm):
        def body(i_vmem, o_vmem):
            pltpu.sync_copy(x_hbm.at[i_vmem.at[0, pl.ds(0, GW)]], o_vmem)

        pltpu.emit_pipeline(
            body,
            grid=(NK * QN // GW,),
            in_specs=[pl.BlockSpec((1, 128), index_map=lambda i: (i, 0))],
            out_specs=[pl.BlockSpec((GW, VDP), index_map=lambda i: (i, 0))],
            core_axis_name=("core", "subcore"),
            dimension_semantics=(pltpu.PARALLEL,),
        )(i_hbm, o_hbm)

    return kern(values, flat_idx)


def _mean_body(g_ref, o_ref):
    acc = g_ref[0:QN, :]
    for j in range(1, NK):
        acc = acc + g_ref[j * QN:(j + 1) * QN, :]
    o_ref[...] = acc[:, :VD] / NK


@jax.jit
def _mean(gathered):
    return pl.pallas_call(
        _mean_body,
        in_specs=[pl.BlockSpec((NK * QN, VDP), lambda: (0, 0))],
        out_specs=pl.BlockSpec((QN, VD), lambda: (0, 0)),
        out_shape=jax.ShapeDtypeStruct((QN, VD), jnp.float32),
    )(gathered)


@jax.jit
def kernel(queries, keys, values):
    keys_padded = jnp.pad(keys, ((0, KPAD - KN), (0, 0)))
    values_padded = jnp.pad(values, ((0, 0), (0, VDP - VD)))
    idx = _topk(queries, keys_padded)
    flat_idx = idx[:, :NK].T.reshape(NK * QN // GW, GW)
    flat_idx = jnp.pad(flat_idx, ((0, 0), (0, 128 - GW)))
    gathered = _sc_gather(values_padded, flat_idx)
    return _mean(gathered)


# speedup vs baseline: 1.6502x; 1.6502x over previous
import functools

import jax
import jax.numpy as jnp
from jax.experimental import pallas as pl
from jax.experimental.pallas import tpu as pltpu
from jax.experimental.pallas import tpu_sc as plsc

QN = 1024
KD = 16
KN = 100000
VD = 978
NK = 5

KPAD = 102400
BK = 2048
NBLK = KPAD // BK

FMAX = float(3.402823466e38)
IMAX = 2**31 - 1

GW = 16
VDA = 896
VDT = 128


def _topk_body(q_ref, kb_ref, vals_ref, idx_ref):
    b = pl.program_id(0)

    @pl.when(b == 0)
    def _init():
        vals_ref[...] = jnp.full((QN, 16), FMAX, jnp.float32)
        idx_ref[...] = jnp.full((QN, 16), IMAX, jnp.int32)

    q = q_ref[...]
    kb = kb_ref[...]
    qk = jax.lax.dot_general(
        q, kb, (((1,), (1,)), ((), ())),
        preferred_element_type=jnp.float32)
    qsq = jnp.sum(q * q, axis=1, keepdims=True)
    ksq = jnp.sum(kb * kb, axis=1)
    d2 = (qsq - 2.0 * qk) + ksq[None, :]
    gidx = b * BK + jax.lax.broadcasted_iota(jnp.int32, (QN, BK), 1)
    d2 = jnp.where(gidx >= KN, FMAX, d2)

    s = d2
    for j in range(NK):
        m = jnp.min(s, axis=1, keepdims=True)
        am = jnp.min(jnp.where(s == m, gidx, IMAX), axis=1,
                     keepdims=True)
        vals_ref[:, 8 + j:9 + j] = m
        idx_ref[:, 8 + j:9 + j] = am
        if j + 1 < NK:
            s = jnp.where(gidx == am, FMAX, s)

    cv = vals_ref[...]
    ci = idx_ref[...]
    nv = []
    ni = []
    for j in range(NK):
        m = jnp.min(cv, axis=1, keepdims=True)
        am = jnp.min(jnp.where(cv == m, ci, IMAX), axis=1, keepdims=True)
        nv.append(m)
        ni.append(am)
        if j + 1 < NK:
            cv = jnp.where(ci == am, FMAX, cv)
    vals_ref[:, 0:NK] = jnp.concatenate(nv, axis=1)
    idx_ref[:, 0:NK] = jnp.concatenate(ni, axis=1)


@jax.jit
def _topk(queries, keys_padded):
    vals, idx = pl.pallas_call(
        _topk_body,
        grid=(NBLK,),
        in_specs=[
            pl.BlockSpec((QN, KD), lambda i: (0, 0)),
            pl.BlockSpec((BK, KD), lambda i: (i, 0)),
        ],
        out_specs=[
            pl.BlockSpec((QN, 16), lambda i: (0, 0)),
            pl.BlockSpec((QN, 16), lambda i: (0, 0)),
        ],
        out_shape=[
            jax.ShapeDtypeStruct((QN, 16), jnp.float32),
            jax.ShapeDtypeStruct((QN, 16), jnp.int32),
        ],
        compiler_params=pltpu.CompilerParams(
            dimension_semantics=("arbitrary",)),
    )(queries, keys_padded)
    del vals
    return idx


TAIL_BR = 5000


def _tail_body(x_hbm, o_ref, buf, sem):
    b = pl.program_id(0)
    cp = pltpu.make_async_copy(
        x_hbm.at[pl.ds(b * TAIL_BR, TAIL_BR), pl.ds(VDA, VD - VDA)],
        buf, sem)
    cp.start()
    cp.wait()
    o_ref[:, 0:VD - VDA] = buf[...]


@jax.jit
def _tail(values):
    return pl.pallas_call(
        _tail_body,
        grid=(KN // TAIL_BR,),
        in_specs=[pl.BlockSpec(memory_space=pl.ANY)],
        out_specs=pl.BlockSpec((TAIL_BR, VDT), lambda i: (i, 0)),
        out_shape=jax.ShapeDtypeStruct((KN, VDT), jnp.float32),
        scratch_shapes=[pltpu.VMEM((TAIL_BR, VD - VDA), jnp.float32),
                        pltpu.SemaphoreType.DMA],
    )(values)


def _sc_gather(values, tail, flat_idx):
    mesh = plsc.VectorSubcoreMesh(core_axis_name="core",
                                  subcore_axis_name="subcore")

    @pl.kernel(out_type=[jax.ShapeDtypeStruct((NK * QN, VDA), jnp.float32),
                         jax.ShapeDtypeStruct((NK * QN, VDT), jnp.float32)],
               mesh=mesh)
    def kern(x_hbm, t_hbm, i_hbm, om_hbm, ot_hbm):
        def body(i_vmem, om_vmem, ot_vmem):
            rows = i_vmem.at[0, pl.ds(0, GW)]
            pltpu.sync_copy(x_hbm.at[rows, pl.ds(0, VDA)], om_vmem)
            pltpu.sync_copy(t_hbm.at[rows], ot_vmem)

        pltpu.emit_pipeline(
            body,
            grid=(NK * QN // GW,),
            in_specs=[pl.BlockSpec((1, 128), index_map=lambda i: (i, 0))],
            out_specs=[pl.BlockSpec((GW, VDA), index_map=lambda i: (i, 0)),
                       pl.BlockSpec((GW, VDT), index_map=lambda i: (i, 0))],
            core_axis_name=("core", "subcore"),
            dimension_semantics=(pltpu.PARALLEL,),
        )(i_hbm, om_hbm, ot_hbm)

    return kern(values, tail, flat_idx)


def _mean_body(gm_ref, gt_ref, o_ref):
    accm = gm_ref[0:QN, :]
    acct = gt_ref[0:QN, :]
    for j in range(1, NK):
        accm = accm + gm_ref[j * QN:(j + 1) * QN, :]
        acct = acct + gt_ref[j * QN:(j + 1) * QN, :]
    o_ref[:, 0:VDA] = accm / NK
    o_ref[:, VDA:VD] = acct[:, :VD - VDA] / NK


@jax.jit
def _mean(g_main, g_tail):
    return pl.pallas_call(
        _mean_body,
        in_specs=[pl.BlockSpec((NK * QN, VDA), lambda: (0, 0)),
                  pl.BlockSpec((NK * QN, VDT), lambda: (0, 0))],
        out_specs=pl.BlockSpec((QN, VD), lambda: (0, 0)),
        out_shape=jax.ShapeDtypeStruct((QN, VD), jnp.float32),
    )(g_main, g_tail)


@jax.jit
def kernel(queries, keys, values):
    keys_padded = jnp.pad(keys, ((0, KPAD - KN), (0, 0)))
    tail = _tail(values)
    idx = _topk(queries, keys_padded)
    flat_idx = idx[:, :NK].T.reshape(NK * QN // GW, GW)
    flat_idx = jnp.pad(flat_idx, ((0, 0), (0, 128 - GW)))
    g_main, g_tail = _sc_gather(values, tail, flat_idx)
    return _mean(g_main, g_tail)


# speedup vs baseline: 1.7994x; 1.0904x over previous
import functools

import jax
import jax.numpy as jnp
from jax.experimental import pallas as pl
from jax.experimental.pallas import tpu as pltpu
from jax.experimental.pallas import tpu_sc as plsc

QN = 1024
KD = 16
KN = 100000
VD = 978
NK = 5

KPAD = 102400
BK = 2048
NBLK = KPAD // BK

FMAX = float(3.402823466e38)
IMAX = 2**31 - 1

GW = 16
VDA = 896
VDT = 128


def _topk_body(q_ref, kb_ref, vals_ref, idx_ref):
    b = pl.program_id(0)

    @pl.when(b == 0)
    def _init():
        vals_ref[...] = jnp.full((QN, 16), FMAX, jnp.float32)
        idx_ref[...] = jnp.full((QN, 16), FMAX, jnp.float32)

    q = q_ref[...]
    kb = kb_ref[...]
    qk2 = jax.lax.dot_general(
        q * -2.0, kb, (((1,), (1,)), ((), ())),
        preferred_element_type=jnp.float32)
    qsq = jnp.sum(q * q, axis=1, keepdims=True)
    ksq = jnp.sum(kb * kb, axis=1)
    lane = b * BK + jax.lax.broadcasted_iota(jnp.int32, (1, BK), 1)
    ksq_m = jnp.where(lane >= KN, FMAX, ksq[None, :])
    fi = lane.astype(jnp.float32)
    d2 = (qsq + qk2) + ksq_m

    s = d2
    for j in range(NK):
        m = jnp.min(s, axis=1, keepdims=True)
        am = jnp.min(jnp.where(s == m, fi, FMAX), axis=1,
                     keepdims=True)
        vals_ref[:, 8 + j:9 + j] = m
        idx_ref[:, 8 + j:9 + j] = am
        if j + 1 < NK:
            s = jnp.where(fi == am, FMAX, s)

    cv = vals_ref[...]
    ci = idx_ref[...]
    nv = []
    ni = []
    for j in range(NK):
        m = jnp.min(cv, axis=1, keepdims=True)
        am = jnp.min(jnp.where(cv == m, ci, FMAX), axis=1, keepdims=True)
        nv.append(m)
        ni.append(am)
        if j + 1 < NK:
            cv = jnp.where(ci == am, FMAX, cv)
    vals_ref[:, 0:NK] = jnp.concatenate(nv, axis=1)
    idx_ref[:, 0:NK] = jnp.concatenate(ni, axis=1)


@jax.jit
def _topk(queries, keys_padded):
    vals, idx = pl.pallas_call(
        _topk_body,
        grid=(NBLK,),
        in_specs=[
            pl.BlockSpec((QN, KD), lambda i: (0, 0)),
            pl.BlockSpec((BK, KD), lambda i: (i, 0)),
        ],
        out_specs=[
            pl.BlockSpec((QN, 16), lambda i: (0, 0)),
            pl.BlockSpec((QN, 16), lambda i: (0, 0)),
        ],
        out_shape=[
            jax.ShapeDtypeStruct((QN, 16), jnp.float32),
            jax.ShapeDtypeStruct((QN, 16), jnp.float32),
        ],
        compiler_params=pltpu.CompilerParams(
            dimension_semantics=("arbitrary",)),
    )(queries, keys_padded)
    del vals
    return idx


TAIL_BR = 2000


def _tail_body(x_ref, o_ref):
    o_ref[:, 0:VD - VDA] = x_ref[:, VDA:VD]


@jax.jit
def _tail(values):
    return pl.pallas_call(
        _tail_body,
        grid=(KN // TAIL_BR,),
        in_specs=[pl.BlockSpec((TAIL_BR, VD), lambda i: (i, 0))],
        out_specs=pl.BlockSpec((TAIL_BR, VDT), lambda i: (i, 0)),
        out_shape=jax.ShapeDtypeStruct((KN, VDT), jnp.float32),
    )(values)


def _sc_gather(values, tail, flat_idx):
    mesh = plsc.VectorSubcoreMesh(core_axis_name="core",
                                  subcore_axis_name="subcore")

    @pl.kernel(out_type=[jax.ShapeDtypeStruct((NK * QN, VDA), jnp.float32),
                         jax.ShapeDtypeStruct((NK * QN, VDT), jnp.float32)],
               mesh=mesh)
    def kern(x_hbm, t_hbm, i_hbm, om_hbm, ot_hbm):
        def body(i_vmem, om_vmem, ot_vmem):
            rows = i_vmem.at[0, pl.ds(0, GW)]
            pltpu.sync_copy(x_hbm.at[rows, pl.ds(0, VDA)], om_vmem)
            pltpu.sync_copy(t_hbm.at[rows], ot_vmem)

        pltpu.emit_pipeline(
            body,
            grid=(NK * QN // GW,),
            in_specs=[pl.BlockSpec((1, 128), index_map=lambda i: (i, 0))],
            out_specs=[pl.BlockSpec((GW, VDA), index_map=lambda i: (i, 0)),
                       pl.BlockSpec((GW, VDT), index_map=lambda i: (i, 0))],
            core_axis_name=("core", "subcore"),
            dimension_semantics=(pltpu.PARALLEL,),
        )(i_hbm, om_hbm, ot_hbm)

    return kern(values, tail, flat_idx)


def _mean_body(gm_ref, gt_ref, o_ref):
    accm = gm_ref[0:QN, :]
    acct = gt_ref[0:QN, :]
    for j in range(1, NK):
        accm = accm + gm_ref[j * QN:(j + 1) * QN, :]
        acct = acct + gt_ref[j * QN:(j + 1) * QN, :]
    o_ref[:, 0:VDA] = accm / NK
    o_ref[:, VDA:VD] = acct[:, :VD - VDA] / NK


@jax.jit
def _mean(g_main, g_tail):
    return pl.pallas_call(
        _mean_body,
        in_specs=[pl.BlockSpec((NK * QN, VDA), lambda: (0, 0)),
                  pl.BlockSpec((NK * QN, VDT), lambda: (0, 0))],
        out_specs=pl.BlockSpec((QN, VD), lambda: (0, 0)),
        out_shape=jax.ShapeDtypeStruct((QN, VD), jnp.float32),
    )(g_main, g_tail)


@jax.jit
def kernel(queries, keys, values):
    keys_padded = jnp.pad(keys, ((0, KPAD - KN), (0, 0)))
    tail = _tail(values)
    idx = _topk(queries, keys_padded).astype(jnp.int32)
    flat_idx = idx[:, :NK].T.reshape(NK * QN // GW, GW)
    flat_idx = jnp.pad(flat_idx, ((0, 0), (0, 128 - GW)))
    g_main, g_tail = _sc_gather(values, tail, flat_idx)
    return _mean(g_main, g_tail)


# speedup vs baseline: 2.0254x; 1.1256x over previous
import functools

import jax
import jax.numpy as jnp
from jax.experimental import pallas as pl
from jax.experimental.pallas import tpu as pltpu
from jax.experimental.pallas import tpu_sc as plsc

QN = 1024
KD = 16
KN = 100000
VD = 978
NK = 5

KPAD = 102400
BK = 2048
NBLK = KPAD // BK

FMAX = float(3.402823466e38)
IMAX = 2**31 - 1

GW = 16
VDA = 896
VDT = 128


def _topk_body(q_ref, kb_ref, v_ref, vals_ref, idx_ref, tail_ref):
    b = pl.program_id(0)

    tail_ref[:, 0:VD - VDA] = v_ref[:, VDA:VD]

    @pl.when(b == 0)
    def _init():
        vals_ref[...] = jnp.full((QN, 16), FMAX, jnp.float32)
        idx_ref[...] = jnp.full((QN, 16), FMAX, jnp.float32)

    q = q_ref[...]
    kb = kb_ref[...]
    qk2 = jax.lax.dot_general(
        q * -2.0, kb, (((1,), (1,)), ((), ())),
        preferred_element_type=jnp.float32)
    qsq = jnp.sum(q * q, axis=1, keepdims=True)
    ksq = jnp.sum(kb * kb, axis=1)
    lane = b * BK + jax.lax.broadcasted_iota(jnp.int32, (1, BK), 1)
    ksq_m = jnp.where(lane >= KN, FMAX, ksq[None, :])
    fi = lane.astype(jnp.float32)
    d2 = (qsq + qk2) + ksq_m

    s = d2
    for j in range(NK):
        m = jnp.min(s, axis=1, keepdims=True)
        am = jnp.min(jnp.where(s == m, fi, FMAX), axis=1,
                     keepdims=True)
        vals_ref[:, 8 + j:9 + j] = m
        idx_ref[:, 8 + j:9 + j] = am
        if j + 1 < NK:
            s = jnp.where(fi == am, FMAX, s)

    cv = vals_ref[...]
    ci = idx_ref[...]
    nv = []
    ni = []
    for j in range(NK):
        m = jnp.min(cv, axis=1, keepdims=True)
        am = jnp.min(jnp.where(cv == m, ci, FMAX), axis=1, keepdims=True)
        nv.append(m)
        ni.append(am)
        if j + 1 < NK:
            cv = jnp.where(ci == am, FMAX, cv)
    vals_ref[:, 0:NK] = jnp.concatenate(nv, axis=1)
    idx_ref[:, 0:NK] = jnp.concatenate(ni, axis=1)


@jax.jit
def _topk(queries, keys_padded, values):
    vals, idx, tail = pl.pallas_call(
        _topk_body,
        grid=(NBLK,),
        in_specs=[
            pl.BlockSpec((QN, KD), lambda i: (0, 0)),
            pl.BlockSpec((BK, KD), lambda i: (i, 0)),
            pl.BlockSpec((KN // NBLK, VD), lambda i: (i, 0)),
        ],
        out_specs=[
            pl.BlockSpec((QN, 16), lambda i: (0, 0)),
            pl.BlockSpec((QN, 16), lambda i: (0, 0)),
            pl.BlockSpec((KN // NBLK, VDT), lambda i: (i, 0)),
        ],
        out_shape=[
            jax.ShapeDtypeStruct((QN, 16), jnp.float32),
            jax.ShapeDtypeStruct((QN, 16), jnp.float32),
            jax.ShapeDtypeStruct((KN, VDT), jnp.float32),
        ],
        compiler_params=pltpu.CompilerParams(
            dimension_semantics=("arbitrary",)),
    )(queries, keys_padded, values)
    del vals
    return idx, tail


def _sc_gather(values, tail, flat_idx):
    mesh = plsc.VectorSubcoreMesh(core_axis_name="core",
                                  subcore_axis_name="subcore")

    @pl.kernel(out_type=[jax.ShapeDtypeStruct((NK * QN, VDA), jnp.float32),
                         jax.ShapeDtypeStruct((NK * QN, VDT), jnp.float32)],
               mesh=mesh)
    def kern(x_hbm, t_hbm, i_hbm, om_hbm, ot_hbm):
        def body(i_vmem, om_vmem, ot_vmem):
            rows = i_vmem.at[0, pl.ds(0, GW)]
            pltpu.sync_copy(x_hbm.at[rows, pl.ds(0, VDA)], om_vmem)
            pltpu.sync_copy(t_hbm.at[rows], ot_vmem)

        pltpu.emit_pipeline(
            body,
            grid=(NK * QN // GW,),
            in_specs=[pl.BlockSpec((1, 128), index_map=lambda i: (i, 0))],
            out_specs=[pl.BlockSpec((GW, VDA), index_map=lambda i: (i, 0)),
                       pl.BlockSpec((GW, VDT), index_map=lambda i: (i, 0))],
            core_axis_name=("core", "subcore"),
            dimension_semantics=(pltpu.PARALLEL,),
        )(i_hbm, om_hbm, ot_hbm)

    return kern(values, tail, flat_idx)


def _mean_body(gm_ref, gt_ref, o_ref):
    accm = gm_ref[0:QN, :]
    acct = gt_ref[0:QN, :]
    for j in range(1, NK):
        accm = accm + gm_ref[j * QN:(j + 1) * QN, :]
        acct = acct + gt_ref[j * QN:(j + 1) * QN, :]
    o_ref[:, 0:VDA] = accm / NK
    o_ref[:, VDA:VD] = acct[:, :VD - VDA] / NK


@jax.jit
def _mean(g_main, g_tail):
    return pl.pallas_call(
        _mean_body,
        in_specs=[pl.BlockSpec((NK * QN, VDA), lambda: (0, 0)),
                  pl.BlockSpec((NK * QN, VDT), lambda: (0, 0))],
        out_specs=pl.BlockSpec((QN, VD), lambda: (0, 0)),
        out_shape=jax.ShapeDtypeStruct((QN, VD), jnp.float32),
    )(g_main, g_tail)


@jax.jit
def kernel(queries, keys, values):
    keys_padded = jnp.pad(keys, ((0, KPAD - KN), (0, 0)))
    idxf, tail = _topk(queries, keys_padded, values)
    idx = idxf.astype(jnp.int32)
    flat_idx = idx[:, :NK].T.reshape(NK * QN // GW, GW)
    flat_idx = jnp.pad(flat_idx, ((0, 0), (0, 128 - GW)))
    g_main, g_tail = _sc_gather(values, tail, flat_idx)
    return _mean(g_main, g_tail)


# speedup vs baseline: 2.0536x; 1.0139x over previous
import functools

import jax
import jax.numpy as jnp
from jax.experimental import pallas as pl
from jax.experimental.pallas import tpu as pltpu
from jax.experimental.pallas import tpu_sc as plsc

QN = 1024
KD = 16
KN = 100000
VD = 978
NK = 5

KPAD = 102400
BK = 2048
NBLK = KPAD // BK

FMAX = float(3.402823466e38)
IMAX = 2**31 - 1

GW = 16
VDA = 896
VDT = 128


def _topk_body(q_ref, kb_ref, v_ref, vals_ref, idx_ref, tail_ref):
    b = pl.program_id(0)

    tail_ref[:, 0:VD - VDA] = v_ref[:, 0:VD - VDA]

    @pl.when(b == 0)
    def _init():
        vals_ref[...] = jnp.full((QN, 16), FMAX, jnp.float32)
        idx_ref[...] = jnp.full((QN, 16), FMAX, jnp.float32)

    q = q_ref[...]
    kb = kb_ref[...]
    qk2 = jax.lax.dot_general(
        q * -2.0, kb, (((1,), (1,)), ((), ())),
        preferred_element_type=jnp.float32)
    qsq = jnp.sum(q * q, axis=1, keepdims=True)
    ksq = jnp.sum(kb * kb, axis=1)
    lane = b * BK + jax.lax.broadcasted_iota(jnp.int32, (1, BK), 1)
    ksq_m = jnp.where(lane >= KN, FMAX, ksq[None, :])
    fi = lane.astype(jnp.float32)
    d2 = (qsq + qk2) + ksq_m

    s = d2
    for j in range(NK):
        m = jnp.min(s, axis=1, keepdims=True)
        am = jnp.min(jnp.where(s == m, fi, FMAX), axis=1,
                     keepdims=True)
        vals_ref[:, 8 + j:9 + j] = m
        idx_ref[:, 8 + j:9 + j] = am
        if j + 1 < NK:
            s = jnp.where(fi == am, FMAX, s)

    cv = vals_ref[...]
    ci = idx_ref[...]
    nv = []
    ni = []
    for j in range(NK):
        m = jnp.min(cv, axis=1, keepdims=True)
        am = jnp.min(jnp.where(cv == m, ci, FMAX), axis=1, keepdims=True)
        nv.append(m)
        ni.append(am)
        if j + 1 < NK:
            cv = jnp.where(ci == am, FMAX, cv)
    vals_ref[:, 0:NK] = jnp.concatenate(nv, axis=1)
    idx_ref[:, 0:NK] = jnp.concatenate(ni, axis=1)


@jax.jit
def _topk(queries, keys_padded, values):
    vals, idx, tail = pl.pallas_call(
        _topk_body,
        grid=(NBLK,),
        in_specs=[
            pl.BlockSpec((QN, KD), lambda i: (0, 0)),
            pl.BlockSpec((BK, KD), lambda i: (i, 0)),
            pl.BlockSpec((KN // NBLK, VDA), lambda i: (i, 1)),
        ],
        out_specs=[
            pl.BlockSpec((QN, 16), lambda i: (0, 0)),
            pl.BlockSpec((QN, 16), lambda i: (0, 0)),
            pl.BlockSpec((KN // NBLK, VDT), lambda i: (i, 0)),
        ],
        out_shape=[
            jax.ShapeDtypeStruct((QN, 16), jnp.float32),
            jax.ShapeDtypeStruct((QN, 16), jnp.float32),
            jax.ShapeDtypeStruct((KN, VDT), jnp.float32),
        ],
        compiler_params=pltpu.CompilerParams(
            dimension_semantics=("arbitrary",)),
    )(queries, keys_padded, values)
    del vals
    return idx, tail


def _sc_gather(values, tail, flat_idx):
    mesh = plsc.VectorSubcoreMesh(core_axis_name="core",
                                  subcore_axis_name="subcore")

    @pl.kernel(out_type=[jax.ShapeDtypeStruct((NK * QN, VDA), jnp.float32),
                         jax.ShapeDtypeStruct((NK * QN, VDT), jnp.float32)],
               mesh=mesh)
    def kern(x_hbm, t_hbm, i_hbm, om_hbm, ot_hbm):
        def body(i_vmem, om_vmem, ot_vmem):
            rows = i_vmem.at[0, pl.ds(0, GW)]
            pltpu.sync_copy(x_hbm.at[rows, pl.ds(0, VDA)], om_vmem)
            pltpu.sync_copy(t_hbm.at[rows], ot_vmem)

        pltpu.emit_pipeline(
            body,
            grid=(NK * QN // GW,),
            in_specs=[pl.BlockSpec((1, 128), index_map=lambda i: (i, 0))],
            out_specs=[pl.BlockSpec((GW, VDA), index_map=lambda i: (i, 0)),
                       pl.BlockSpec((GW, VDT), index_map=lambda i: (i, 0))],
            core_axis_name=("core", "subcore"),
            dimension_semantics=(pltpu.PARALLEL,),
        )(i_hbm, om_hbm, ot_hbm)

    return kern(values, tail, flat_idx)


def _mean_body(gm_ref, gt_ref, o_ref):
    accm = gm_ref[0:QN, :]
    acct = gt_ref[0:QN, :]
    for j in range(1, NK):
        accm = accm + gm_ref[j * QN:(j + 1) * QN, :]
        acct = acct + gt_ref[j * QN:(j + 1) * QN, :]
    o_ref[:, 0:VDA] = accm / NK
    o_ref[:, VDA:VD] = acct[:, :VD - VDA] / NK


@jax.jit
def _mean(g_main, g_tail):
    return pl.pallas_call(
        _mean_body,
        in_specs=[pl.BlockSpec((NK * QN, VDA), lambda: (0, 0)),
                  pl.BlockSpec((NK * QN, VDT), lambda: (0, 0))],
        out_specs=pl.BlockSpec((QN, VD), lambda: (0, 0)),
        out_shape=jax.ShapeDtypeStruct((QN, VD), jnp.float32),
    )(g_main, g_tail)


@jax.jit
def kernel(queries, keys, values):
    keys_padded = jnp.pad(keys, ((0, KPAD - KN), (0, 0)))
    idxf, tail = _topk(queries, keys_padded, values)
    idx = idxf.astype(jnp.int32)
    flat_idx = idx[:, :NK].T.reshape(NK * QN // GW, GW)
    flat_idx = jnp.pad(flat_idx, ((0, 0), (0, 128 - GW)))
    g_main, g_tail = _sc_gather(values, tail, flat_idx)
    return _mean(g_main, g_tail)
